# baseline (speedup 1.0000x reference)
import jax
import jax.numpy as jnp
from jax.experimental import pallas as pl
from jax.experimental.pallas import tpu as pltpu

LAMBDA_FAKE_W = 1.0
B = 256
D = 150528
RB = 4
NSTEPS = B // RB
NBUF = 8


def _start_copy(hbm_ref, buf_ref, sem, chunk, slot):
    pltpu.make_async_copy(
        hbm_ref.at[pl.ds(chunk * RB, RB), :],
        buf_ref.at[slot],
        sem.at[slot],
    ).start()


def _wait_copy(hbm_ref, buf_ref, sem, chunk, slot):
    pltpu.make_async_copy(
        hbm_ref.at[pl.ds(chunk * RB, RB), :],
        buf_ref.at[slot],
        sem.at[slot],
    ).wait()


def _loss_kernel(y_ref, r_hbm, x_hbm, o_ref, rbuf, xbuf, rsem, xsem, acc_ref):
    step = pl.program_id(0)

    @pl.when(step == 0)
    def _prologue():
        acc_ref[0] = 0.0
        acc_ref[1] = 0.0
        acc_ref[2] = 0.0
        for b in range(NBUF):
            _start_copy(r_hbm, rbuf, rsem, b, b)
            _start_copy(x_hbm, xbuf, xsem, b, b)

    slot = jax.lax.rem(step, NBUF)
    _wait_copy(r_hbm, rbuf, rsem, step, slot)
    _wait_copy(x_hbm, xbuf, xsem, step, slot)

    d = jnp.abs(rbuf[slot] - xbuf[slot])
    s = jnp.sum(d, axis=1, keepdims=True)
    yv = y_ref[pl.ds(step * RB, RB), :]
    acc_ref[0] += jnp.sum(s * yv)
    acc_ref[1] += jnp.sum(s)
    acc_ref[2] += jnp.sum(yv)

    @pl.when(step + NBUF < NSTEPS)
    def _refill():
        _start_copy(r_hbm, rbuf, rsem, step + NBUF, slot)
        _start_copy(x_hbm, xbuf, xsem, step + NBUF, slot)

    @pl.when(step == NSTEPS - 1)
    def _finalize():
        n_real = acc_ref[2]
        n_fake = B - n_real
        sum_real = acc_ref[0]
        sum_fake = acc_ref[1] - sum_real
        loss_real = jnp.where(n_real > 0, sum_real / (n_real * D), 0.0)
        loss_fake = jnp.where(n_fake > 0, sum_fake / (n_fake * D), 0.0)
        o_ref[...] = (loss_real - LAMBDA_FAKE_W * loss_fake).reshape(1, 1)


def kernel(recons, x, y):
    r2 = recons.reshape(B, D)
    x2 = x.reshape(B, D)
    y2 = y.astype(jnp.float32).reshape(B, 1)

    out = pl.pallas_call(
        _loss_kernel,
        grid=(NSTEPS,),
        in_specs=[
            pl.BlockSpec((B, 1), lambda i: (0, 0)),
            pl.BlockSpec(memory_space=pltpu.MemorySpace.HBM),
            pl.BlockSpec(memory_space=pltpu.MemorySpace.HBM),
        ],
        out_specs=pl.BlockSpec((1, 1), lambda i: (0, 0)),
        out_shape=jax.ShapeDtypeStruct((1, 1), jnp.float32),
        scratch_shapes=[
            pltpu.VMEM((NBUF, RB, D), jnp.float32),
            pltpu.VMEM((NBUF, RB, D), jnp.float32),
            pltpu.SemaphoreType.DMA((NBUF,)),
            pltpu.SemaphoreType.DMA((NBUF,)),
            pltpu.SMEM((3,), jnp.float32),
        ],
        compiler_params=pltpu.CompilerParams(
            dimension_semantics=("arbitrary",),
        ),
    )(y2, r2, x2)
    return out.reshape(())
